# baseline (device time: 95559 ns/iter reference)
import jax
import jax.numpy as jnp
from jax import lax
from jax.experimental import pallas as pl
from jax.experimental.pallas import tpu as pltpu

N_DEV = 16


def kernel(x):
    m, n = x.shape
    blk = n // N_DEV

    def body(x_ref, out_ref, xbuf, stage_ref, in_sems, cp_sem,
             send_sems, recv_sems):
        my = lax.axis_index("i")

        barrier = pltpu.get_barrier_semaphore()
        for off in range(1, N_DEV):
            pl.semaphore_signal(
                barrier, inc=1,
                device_id=((my + off) % N_DEV,),
                device_id_type=pl.DeviceIdType.MESH,
            )
        pl.semaphore_wait(barrier, N_DEV - 1)

        offsets = sorted(range(1, N_DEV), key=lambda o: -min(o, N_DEV - o))

        loads = []
        for idx in range(N_DEV - 1):
            j = (my + offsets[idx]) % N_DEV
            loads.append(pltpu.make_async_copy(
                x_ref.at[:, pl.ds(j * blk, blk)],
                xbuf.at[idx % 2],
                in_sems.at[idx % 2],
            ))
        loads[0].start()

        sends = []
        for idx in range(N_DEV - 1):
            j = (my + offsets[idx]) % N_DEV
            loads[idx].wait()
            if idx + 1 < N_DEV - 1:
                loads[idx + 1].start()
            stage_ref[pl.ds(j, 1), :, :] = (
                xbuf[idx % 2].astype(jnp.bfloat16).reshape(1, m, blk)
            )
            rdma = pltpu.make_async_remote_copy(
                src_ref=stage_ref.at[j],
                dst_ref=out_ref.at[pl.ds(my * blk, blk), :],
                send_sem=send_sems.at[j],
                recv_sem=recv_sems.at[my],
                device_id=(j,),
                device_id_type=pl.DeviceIdType.MESH,
            )
            rdma.start()
            sends.append(rdma)

        own_in = pltpu.make_async_copy(
            x_ref.at[:, pl.ds(my * blk, blk)], xbuf.at[0], in_sems.at[0])
        own_in.start()
        own_in.wait()
        stage_ref[pl.ds(my, 1), :, :] = (
            xbuf[0].astype(jnp.bfloat16).reshape(1, m, blk)
        )
        own_out = pltpu.make_async_copy(
            stage_ref.at[my], out_ref.at[pl.ds(my * blk, blk), :], cp_sem)
        own_out.start()

        for off in range(1, N_DEV):
            j = (my + off) % N_DEV
            recv = pltpu.make_async_remote_copy(
                src_ref=stage_ref.at[j],
                dst_ref=out_ref.at[pl.ds(j * blk, blk), :],
                send_sem=send_sems.at[j],
                recv_sem=recv_sems.at[j],
                device_id=(j,),
                device_id_type=pl.DeviceIdType.MESH,
            )
            recv.wait_recv()

        own_out.wait()
        for rdma in sends:
            rdma.wait_send()

    return pl.pallas_call(
        body,
        out_shape=jax.ShapeDtypeStruct((N_DEV * m, blk), jnp.bfloat16),
        in_specs=[pl.BlockSpec(memory_space=pl.ANY)],
        out_specs=pl.BlockSpec(memory_space=pl.ANY),
        scratch_shapes=[
            pltpu.VMEM((2, m, blk), jnp.float32),
            pltpu.VMEM((N_DEV, m, blk), jnp.bfloat16),
            pltpu.SemaphoreType.DMA((2,)),
            pltpu.SemaphoreType.DMA,
            pltpu.SemaphoreType.DMA((N_DEV,)),
            pltpu.SemaphoreType.DMA((N_DEV,)),
        ],
        compiler_params=pltpu.CompilerParams(collective_id=0),
    )(x)


# device time: 92955 ns/iter; 1.0280x vs baseline; 1.0280x over previous
import jax
import jax.numpy as jnp
from jax import lax
from jax.experimental import pallas as pl
from jax.experimental.pallas import tpu as pltpu

N_DEV = 16


def kernel(x):
    m, n = x.shape
    blk = n // N_DEV

    def body(x_ref, out_ref, xbuf, stage_ref, in_sems, cp_sem,
             send_sems, recv_sems):
        my = lax.axis_index("i")

        barrier = pltpu.get_barrier_semaphore()
        for off in range(1, N_DEV):
            pl.semaphore_signal(
                barrier, inc=1,
                device_id=((my + off) % N_DEV,),
                device_id_type=pl.DeviceIdType.MESH,
            )
        pl.semaphore_wait(barrier, N_DEV - 1)

        offsets = list(range(1, N_DEV))

        loads = []
        for idx in range(N_DEV - 1):
            j = (my + offsets[idx]) % N_DEV
            loads.append(pltpu.make_async_copy(
                x_ref.at[:, pl.ds(j * blk, blk)],
                xbuf.at[idx % 2],
                in_sems.at[idx % 2],
            ))
        loads[0].start()

        sends = []
        for idx in range(N_DEV - 1):
            j = (my + offsets[idx]) % N_DEV
            loads[idx].wait()
            if idx + 1 < N_DEV - 1:
                loads[idx + 1].start()
            stage_ref[pl.ds(j, 1), :, :] = (
                xbuf[idx % 2].astype(jnp.bfloat16).reshape(1, m, blk)
            )
            rdma = pltpu.make_async_remote_copy(
                src_ref=stage_ref.at[j],
                dst_ref=out_ref.at[pl.ds(my * blk, blk), :],
                send_sem=send_sems.at[j],
                recv_sem=recv_sems.at[my],
                device_id=(j,),
                device_id_type=pl.DeviceIdType.MESH,
            )
            rdma.start()
            sends.append(rdma)

        own_in = pltpu.make_async_copy(
            x_ref.at[:, pl.ds(my * blk, blk)], xbuf.at[0], in_sems.at[0])
        own_in.start()
        own_in.wait()
        stage_ref[pl.ds(my, 1), :, :] = (
            xbuf[0].astype(jnp.bfloat16).reshape(1, m, blk)
        )
        own_out = pltpu.make_async_copy(
            stage_ref.at[my], out_ref.at[pl.ds(my * blk, blk), :], cp_sem)
        own_out.start()

        for off in range(1, N_DEV):
            j = (my + off) % N_DEV
            recv = pltpu.make_async_remote_copy(
                src_ref=stage_ref.at[j],
                dst_ref=out_ref.at[pl.ds(j * blk, blk), :],
                send_sem=send_sems.at[j],
                recv_sem=recv_sems.at[j],
                device_id=(j,),
                device_id_type=pl.DeviceIdType.MESH,
            )
            recv.wait_recv()

        own_out.wait()
        for rdma in sends:
            rdma.wait_send()

    return pl.pallas_call(
        body,
        out_shape=jax.ShapeDtypeStruct((N_DEV * m, blk), jnp.bfloat16),
        in_specs=[pl.BlockSpec(memory_space=pl.ANY)],
        out_specs=pl.BlockSpec(memory_space=pl.ANY),
        scratch_shapes=[
            pltpu.VMEM((2, m, blk), jnp.float32),
            pltpu.VMEM((N_DEV, m, blk), jnp.bfloat16),
            pltpu.SemaphoreType.DMA((2,)),
            pltpu.SemaphoreType.DMA,
            pltpu.SemaphoreType.DMA((N_DEV,)),
            pltpu.SemaphoreType.DMA((N_DEV,)),
        ],
        compiler_params=pltpu.CompilerParams(collective_id=0),
    )(x)
